# baseline (speedup 1.0000x reference)
import functools

import jax
import jax.numpy as jnp
from jax import lax
from jax.experimental import pallas as pl
from jax.experimental.pallas import tpu as pltpu
from jax.experimental.pallas import tpu_sc as plsc

_NC = 2
_NS = 16
_NW = _NC * _NS
_K = 128
_NBUF = 4


def _emb_body(idx_hbm, table_hbm, out_hbm, idx_v, rows_v, *gsems):
    wid = lax.axis_index("s") * _NC + lax.axis_index("c")
    chunks = idx_hbm.shape[1]
    d = table_hbm.shape[1]
    pltpu.sync_copy(idx_hbm.at[wid], idx_v)
    row0 = wid * chunks * _K

    @pl.loop(0, chunks, step=_NBUF)
    def _group(g0):
        cps = []
        for b in range(_NBUF):
            g = g0 + b
            cps.append(
                pltpu.async_copy(table_hbm.at[idx_v.at[g]], rows_v.at[b], gsems[b])
            )
        for b in range(_NBUF):
            cps[b].wait()
            pltpu.sync_copy(
                rows_v.at[b], out_hbm.at[pl.ds(row0 + (g0 + b) * _K, _K)]
            )


def kernel(x, weight):
    b, s = x.shape
    v, d = weight.shape
    n = b * s
    assert n % (_NW * _K) == 0
    chunks = n // (_NW * _K)
    flat = x.reshape(-1).astype(jnp.int32)
    idx3 = flat.reshape(_NW, chunks, _K)

    mesh = plsc.VectorSubcoreMesh(core_axis_name="c", subcore_axis_name="s")
    run = pl.kernel(
        _emb_body,
        out_type=jax.ShapeDtypeStruct((n, d), jnp.float32),
        mesh=mesh,
        scratch_types=[
            pltpu.VMEM((chunks, _K), jnp.int32),
            pltpu.VMEM((_NBUF, _K, d), jnp.float32),
        ]
        + [pltpu.SemaphoreType.DMA] * _NBUF,
        compiler_params=pltpu.CompilerParams(use_tc_tiling_on_sc=False),
    )
    out = run(idx3, weight)
    return out.reshape(b, s, d)


# speedup vs baseline: 1.0803x; 1.0803x over previous
import functools

import jax
import jax.numpy as jnp
from jax import lax
from jax.experimental import pallas as pl
from jax.experimental.pallas import tpu as pltpu
from jax.experimental.pallas import tpu_sc as plsc

_NC = 2
_NS = 16
_NW = _NC * _NS
_K = 128
_NBUF = 4


def _emb_body(idx_hbm, table_hbm, out_hbm, idx_v, rows_v, *gsems):
    wid = lax.axis_index("s") * _NC + lax.axis_index("c")
    chunks = idx_hbm.shape[1]
    d = table_hbm.shape[1]
    pltpu.sync_copy(idx_hbm.at[wid], idx_v)
    row0 = wid * chunks * _K

    @pl.loop(0, chunks, step=_NBUF)
    def _group(g0):
        cps = []
        for b in range(_NBUF):
            g = g0 + b
            cps.append(
                pltpu.async_copy(table_hbm.at[idx_v.at[g]], rows_v.at[b], gsems[b])
            )
        for b in range(_NBUF):
            cps[b].wait()
            pltpu.sync_copy(
                rows_v.at[b], out_hbm.at[pl.ds(row0 + (g0 + b) * _K, _K)]
            )


def kernel(x, weight):
    b, s = x.shape
    v, d = weight.shape
    n = b * s
    assert n % (_NW * _K) == 0
    chunks = n // (_NW * _K)
    wpad = jnp.pad(weight, ((0, 0), (0, 128 - d))).reshape(2 * v, d)
    flat = x.reshape(-1).astype(jnp.int32) * 2
    idx3 = flat.reshape(_NW, chunks, _K)

    mesh = plsc.VectorSubcoreMesh(core_axis_name="c", subcore_axis_name="s")
    run = pl.kernel(
        _emb_body,
        out_type=jax.ShapeDtypeStruct((n, d), jnp.float32),
        mesh=mesh,
        scratch_types=[
            pltpu.VMEM((chunks, _K), jnp.int32),
            pltpu.VMEM((_NBUF, _K, d), jnp.float32),
        ]
        + [pltpu.SemaphoreType.DMA] * _NBUF,
        compiler_params=pltpu.CompilerParams(use_tc_tiling_on_sc=False),
    )
    out = run(idx3, wpad)
    return out.reshape(b, s, d)
